# baseline (speedup 1.0000x reference)
import functools

import jax
import jax.numpy as jnp
from jax import lax
from jax.experimental import pallas as pl
from jax.experimental.pallas import tpu as pltpu
from jax.experimental.pallas import tpu_sc as plsc

B = 64
DM = 1024
VOCAB_N = 100000
K = 50
GW = 128
VC = 4096
NSTEP = 25
VPAD = NSTEP * VC
NG = VPAD // GW
GPS = VC // GW
BIGI = 2**30


def _sc_gather_rows(table, idxs, rows_per_worker, workers):
    info = plsc.get_sparse_core_info()
    nc = info.num_cores
    n, w = idxs.shape[0], table.shape[1]
    assert n == rows_per_worker * workers and rows_per_worker % 8 == 0
    mesh = plsc.VectorSubcoreMesh(core_axis_name="c", subcore_axis_name="s")

    @functools.partial(
        pl.kernel,
        mesh=mesh,
        out_type=jax.ShapeDtypeStruct((n, w), jnp.float32),
        scratch_types=[
            pltpu.VMEM((rows_per_worker,), jnp.int32),
            pltpu.VMEM((rows_per_worker, w), jnp.float32),
            pltpu.SemaphoreType.DMA,
        ],
    )
    def k(table_hbm, idx_hbm, out_hbm, idx_v, rows_v, sem):
        wid = lax.axis_index("s") * nc + lax.axis_index("c")

        @pl.when(wid < workers)
        def _():
            base = wid * rows_per_worker
            pltpu.sync_copy(idx_hbm.at[pl.ds(base, rows_per_worker)], idx_v)
            pltpu.async_copy(table_hbm.at[idx_v], rows_v, sem).wait()
            pltpu.sync_copy(rows_v, out_hbm.at[pl.ds(base, rows_per_worker)])

    return k(table, idxs)


def _matmul_select(x, head):
    def body(x_ref, h_ref, logits_ref, topgf_ref, gm_ref):
        j = pl.program_id(0)
        lg = jnp.dot(x_ref[...], h_ref[...],
                     preferred_element_type=jnp.float32)
        logits_ref[...] = lg
        col = lax.broadcasted_iota(jnp.int32, (B, VC), 1) + j * VC
        lgm = jnp.where(col < VOCAB_N, lg, -jnp.inf)
        gmax = jnp.max(lgm.reshape(B, GPS, GW), axis=-1)
        gm_ref[pl.ds(j * GPS, GPS), :] = gmax.T

        @pl.when(j == NSTEP - 1)
        def _():
            gidv = lax.broadcasted_iota(jnp.int32, (NG, B), 0)
            klane = lax.broadcasted_iota(jnp.int32, (B, K), 1)

            def sel(k, carry):
                gm, topg = carry
                m = jnp.max(gm, axis=0, keepdims=True)
                gid = jnp.min(jnp.where(gm == m, gidv, BIGI), axis=0)
                topg = jnp.where(klane == k, gid[:, None], topg)
                gm = jnp.where(gidv == gid[None, :], -jnp.inf, gm)
                return gm, topg

            _, topg = lax.fori_loop(
                0, K, sel,
                (gm_ref[...], jnp.zeros((B, K), jnp.int32)))
            row = lax.broadcasted_iota(jnp.int32, (B, K), 0)
            topgf_ref[...] = topg + row * NG

    return pl.pallas_call(
        body,
        grid=(NSTEP,),
        in_specs=[
            pl.BlockSpec((B, DM), lambda j: (0, 0)),
            pl.BlockSpec((DM, VC), lambda j: (0, j)),
        ],
        out_specs=[
            pl.BlockSpec((B, VC), lambda j: (0, j)),
            pl.BlockSpec((B, K), lambda j: (0, 0)),
        ],
        out_shape=[
            jax.ShapeDtypeStruct((B, VPAD), jnp.float32),
            jax.ShapeDtypeStruct((B, K), jnp.int32),
        ],
        scratch_shapes=[pltpu.VMEM((NG, B), jnp.float32)],
    )(x, head)


def _finalize(cand, topgf, gnoise):
    C = K * GW
    PH = 10

    def body(cand_ref, topgf_ref, g_ref, next_ref, probs_ref, topi_ref,
             cs_ref):
        tgf = topgf_ref[...]
        rk = lax.broadcasted_iota(jnp.int32, (B, K, GW), 0)
        tg3 = tgf[:, :, None] - rk * NG
        vid3 = tg3 * GW + lax.broadcasted_iota(jnp.int32, (B, K, GW), 2)
        vid = vid3.reshape(B, C)
        cs_ref[...] = jnp.where(vid < VOCAB_N, cand_ref[...], -jnp.inf)
        klane = lax.broadcasted_iota(jnp.int32, (B, K), 1)

        tv = jnp.zeros((B, K), jnp.float32)
        ti = jnp.zeros((B, K), jnp.int32)
        for ph in range(K // PH):
            W = (ph + 1) * PH * GW
            vidp = vid[:, :W]

            def ext(k, carry, vidp=vidp):
                cp, tv, ti = carry
                m = jnp.max(cp, axis=1, keepdims=True)
                wv = jnp.min(jnp.where(cp == m, vidp, BIGI), axis=1)
                tv = jnp.where(klane == k, m, tv)
                ti = jnp.where(klane == k, wv[:, None], ti)
                cp = jnp.where(vidp == wv[:, None], -jnp.inf, cp)
                return cp, tv, ti

            cp, tv, ti = lax.fori_loop(
                ph * PH, (ph + 1) * PH, ext, (cs_ref[:, :W], tv, ti))
            if (ph + 1) * PH < K:
                cs_ref[:, :W] = cp

        z = tv
        p = jnp.exp(z - z[:, 0:1])
        probs = p / jnp.sum(p, axis=1, keepdims=True)
        probs_ref[...] = probs
        topi_ref[...] = ti
        score = jnp.log(probs + 1e-20) + g_ref[...]
        sm = jnp.max(score, axis=1, keepdims=True)
        ix = jnp.min(jnp.where(score == sm, klane, BIGI), axis=1)
        next_ref[...] = jnp.sum(
            jnp.where(klane == ix[:, None], ti, 0), axis=1)[:, None]

    return pl.pallas_call(
        body,
        out_shape=[
            jax.ShapeDtypeStruct((B, 1), jnp.int32),
            jax.ShapeDtypeStruct((B, K), jnp.float32),
            jax.ShapeDtypeStruct((B, K), jnp.int32),
        ],
        scratch_shapes=[pltpu.VMEM((B, C), jnp.float32)],
    )(cand, topgf, gnoise)


def _sc_relayout(head_flat):
    info = plsc.get_sparse_core_info()
    nc = info.num_cores
    mesh = plsc.VectorSubcoreMesh(core_axis_name="c", subcore_axis_name="s")
    PSZ = DM * VC

    @functools.partial(
        pl.kernel,
        mesh=mesh,
        out_type=jax.ShapeDtypeStruct((NSTEP * PSZ,), jnp.float32),
        scratch_types=[pltpu.VMEM((VOCAB_N,), jnp.float32)],
    )
    def k(head_hbm, out_hbm, buf):
        wid = lax.axis_index("s") * nc + lax.axis_index("c")

        def row(r, c):
            d = wid * 32 + r
            pltpu.sync_copy(head_hbm.at[pl.ds(d * VOCAB_N, VOCAB_N)], buf)
            for j in range(NSTEP - 1):
                pltpu.sync_copy(
                    buf.at[pl.ds(j * VC, VC)],
                    out_hbm.at[pl.ds(j * PSZ + d * VC, VC)])
            pltpu.sync_copy(
                buf.at[pl.ds((NSTEP - 1) * VC, VOCAB_N - (NSTEP - 1) * VC)],
                out_hbm.at[pl.ds((NSTEP - 1) * PSZ + d * VC,
                                 VOCAB_N - (NSTEP - 1) * VC)])
            return c

        lax.fori_loop(0, 32, row, 0)

    return k(head_flat)


def kernel(idx, embed, head):
    last = idx[:, -1].astype(jnp.int32)
    g = jax.random.gumbel(jax.random.key(42), (B, K), jnp.float32)
    x = _sc_gather_rows(embed, last, rows_per_worker=8, workers=8)
    panels = _sc_relayout(head.reshape(-1))
    return (panels[:B, None].astype(jnp.int32), x[:, :K],
            jnp.zeros((B, K), jnp.int32))
    logits, topgf = _matmul_select(x, head)
    cand = _sc_gather_rows(logits.reshape(B * NG, GW), topgf.reshape(B * K),
                           rows_per_worker=128, workers=25)
    nxt, probs, topi = _finalize(cand.reshape(B, K * GW), topgf, g)
    return nxt, probs, topi


# speedup vs baseline: 2.0684x; 2.0684x over previous
import functools

import jax
import jax.numpy as jnp
from jax import lax
from jax.experimental import pallas as pl
from jax.experimental.pallas import tpu as pltpu
from jax.experimental.pallas import tpu_sc as plsc

B = 64
DM = 1024
VOCAB_N = 100000
K = 50
GW = 128
VC = 4096
NSTEP = 25
VPAD = NSTEP * VC
NG = VPAD // GW
GPS = VC // GW
BIGI = 2**30


def _sc_gather_rows(table, idxs, rows_per_worker, workers):
    info = plsc.get_sparse_core_info()
    nc = info.num_cores
    n, w = idxs.shape[0], table.shape[1]
    assert n == rows_per_worker * workers and rows_per_worker % 8 == 0
    mesh = plsc.VectorSubcoreMesh(core_axis_name="c", subcore_axis_name="s")

    @functools.partial(
        pl.kernel,
        mesh=mesh,
        out_type=jax.ShapeDtypeStruct((n, w), jnp.float32),
        scratch_types=[
            pltpu.VMEM((rows_per_worker,), jnp.int32),
            pltpu.VMEM((rows_per_worker, w), jnp.float32),
            pltpu.SemaphoreType.DMA,
        ],
    )
    def k(table_hbm, idx_hbm, out_hbm, idx_v, rows_v, sem):
        wid = lax.axis_index("s") * nc + lax.axis_index("c")

        @pl.when(wid < workers)
        def _():
            base = wid * rows_per_worker
            pltpu.sync_copy(idx_hbm.at[pl.ds(base, rows_per_worker)], idx_v)
            pltpu.async_copy(table_hbm.at[idx_v], rows_v, sem).wait()
            pltpu.sync_copy(rows_v, out_hbm.at[pl.ds(base, rows_per_worker)])

    return k(table, idxs)


def _matmul_select(x, head):
    def body(x_ref, h_ref, logits_ref, topgf_ref, gm_ref):
        j = pl.program_id(0)
        lg = jnp.dot(x_ref[...], h_ref[...],
                     preferred_element_type=jnp.float32)
        logits_ref[...] = lg
        col = lax.broadcasted_iota(jnp.int32, (B, VC), 1) + j * VC
        lgm = jnp.where(col < VOCAB_N, lg, -jnp.inf)
        gmax = jnp.max(lgm.reshape(B, GPS, GW), axis=-1)
        gm_ref[pl.ds(j * GPS, GPS), :] = gmax.T

        @pl.when(j == NSTEP - 1)
        def _():
            gidv = lax.broadcasted_iota(jnp.int32, (NG, B), 0)
            klane = lax.broadcasted_iota(jnp.int32, (B, K), 1)

            def sel(k, carry):
                gm, topg = carry
                m = jnp.max(gm, axis=0, keepdims=True)
                gid = jnp.min(jnp.where(gm == m, gidv, BIGI), axis=0)
                topg = jnp.where(klane == k, gid[:, None], topg)
                gm = jnp.where(gidv == gid[None, :], -jnp.inf, gm)
                return gm, topg

            _, topg = lax.fori_loop(
                0, K, sel,
                (gm_ref[...], jnp.zeros((B, K), jnp.int32)))
            row = lax.broadcasted_iota(jnp.int32, (B, K), 0)
            topgf_ref[...] = topg + row * NG

    return pl.pallas_call(
        body,
        grid=(NSTEP,),
        in_specs=[
            pl.BlockSpec((B, DM), lambda j: (0, 0)),
            pl.BlockSpec((DM, VC), lambda j: (0, j)),
        ],
        out_specs=[
            pl.BlockSpec((B, VC), lambda j: (0, j)),
            pl.BlockSpec((B, K), lambda j: (0, 0)),
        ],
        out_shape=[
            jax.ShapeDtypeStruct((B, VPAD), jnp.float32),
            jax.ShapeDtypeStruct((B, K), jnp.int32),
        ],
        scratch_shapes=[pltpu.VMEM((NG, B), jnp.float32)],
    )(x, head)


def _finalize(cand, topgf, gnoise):
    C = K * GW
    PH = 10

    def body(cand_ref, topgf_ref, g_ref, next_ref, probs_ref, topi_ref,
             cs_ref):
        tgf = topgf_ref[...]
        rk = lax.broadcasted_iota(jnp.int32, (B, K, GW), 0)
        tg3 = tgf[:, :, None] - rk * NG
        vid3 = tg3 * GW + lax.broadcasted_iota(jnp.int32, (B, K, GW), 2)
        vid = vid3.reshape(B, C)
        cs_ref[...] = jnp.where(vid < VOCAB_N, cand_ref[...], -jnp.inf)
        klane = lax.broadcasted_iota(jnp.int32, (B, K), 1)

        tv = jnp.zeros((B, K), jnp.float32)
        ti = jnp.zeros((B, K), jnp.int32)
        for ph in range(K // PH):
            W = (ph + 1) * PH * GW
            vidp = vid[:, :W]

            def ext(k, carry, vidp=vidp):
                cp, tv, ti = carry
                m = jnp.max(cp, axis=1, keepdims=True)
                wv = jnp.min(jnp.where(cp == m, vidp, BIGI), axis=1)
                tv = jnp.where(klane == k, m, tv)
                ti = jnp.where(klane == k, wv[:, None], ti)
                cp = jnp.where(vidp == wv[:, None], -jnp.inf, cp)
                return cp, tv, ti

            cp, tv, ti = lax.fori_loop(
                ph * PH, (ph + 1) * PH, ext, (cs_ref[:, :W], tv, ti))
            if (ph + 1) * PH < K:
                cs_ref[:, :W] = cp

        z = tv
        p = jnp.exp(z - z[:, 0:1])
        probs = p / jnp.sum(p, axis=1, keepdims=True)
        probs_ref[...] = probs
        topi_ref[...] = ti
        score = jnp.log(probs + 1e-20) + g_ref[...]
        sm = jnp.max(score, axis=1, keepdims=True)
        ix = jnp.min(jnp.where(score == sm, klane, BIGI), axis=1)
        next_ref[...] = jnp.sum(
            jnp.where(klane == ix[:, None], ti, 0), axis=1)[:, None]

    return pl.pallas_call(
        body,
        out_shape=[
            jax.ShapeDtypeStruct((B, 1), jnp.int32),
            jax.ShapeDtypeStruct((B, K), jnp.float32),
            jax.ShapeDtypeStruct((B, K), jnp.int32),
        ],
        scratch_shapes=[pltpu.VMEM((B, C), jnp.float32)],
    )(cand, topgf, gnoise)


def kernel(idx, embed, head):
    last = idx[:, -1].astype(jnp.int32)
    g = jax.random.gumbel(jax.random.key(42), (B, K), jnp.float32)
    x = _sc_gather_rows(embed, last, rows_per_worker=8, workers=8)
    logits, topgf = _matmul_select(x, head)
    cand = _sc_gather_rows(logits.reshape(B * NG, GW), topgf.reshape(B * K),
                           rows_per_worker=128, workers=25)
    nxt, probs, topi = _finalize(cand.reshape(B, K * GW), topgf, g)
    return nxt, probs, topi


# speedup vs baseline: 2.0800x; 1.0056x over previous
import functools

import jax
import jax.numpy as jnp
from jax import lax
from jax.experimental import pallas as pl
from jax.experimental.pallas import tpu as pltpu
from jax.experimental.pallas import tpu_sc as plsc

B = 64
DM = 1024
VOCAB_N = 100000
K = 50
GW = 128
VC = 2048
NSTEP = 49
VPAD = NSTEP * VC
NG = VPAD // GW
GPS = VC // GW
BIGI = 2**30


def _sc_gather_rows(table, idxs, rows_per_worker, workers):
    info = plsc.get_sparse_core_info()
    nc = info.num_cores
    n, w = idxs.shape[0], table.shape[1]
    assert n == rows_per_worker * workers and rows_per_worker % 8 == 0
    mesh = plsc.VectorSubcoreMesh(core_axis_name="c", subcore_axis_name="s")

    @functools.partial(
        pl.kernel,
        mesh=mesh,
        out_type=jax.ShapeDtypeStruct((n, w), jnp.float32),
        scratch_types=[
            pltpu.VMEM((rows_per_worker,), jnp.int32),
            pltpu.VMEM((rows_per_worker, w), jnp.float32),
            pltpu.SemaphoreType.DMA,
        ],
    )
    def k(table_hbm, idx_hbm, out_hbm, idx_v, rows_v, sem):
        wid = lax.axis_index("s") * nc + lax.axis_index("c")

        @pl.when(wid < workers)
        def _():
            base = wid * rows_per_worker
            pltpu.sync_copy(idx_hbm.at[pl.ds(base, rows_per_worker)], idx_v)
            pltpu.async_copy(table_hbm.at[idx_v], rows_v, sem).wait()
            pltpu.sync_copy(rows_v, out_hbm.at[pl.ds(base, rows_per_worker)])

    return k(table, idxs)


def _matmul_select(x, head):
    def body(x_ref, h_ref, logits_ref, topgf_ref, gm_ref):
        j = pl.program_id(0)
        lg = jnp.dot(x_ref[...], h_ref[...],
                     preferred_element_type=jnp.float32)
        logits_ref[...] = lg
        col = lax.broadcasted_iota(jnp.int32, (B, VC), 1) + j * VC
        lgm = jnp.where(col < VOCAB_N, lg, -jnp.inf)
        gmax = jnp.max(lgm.reshape(B, GPS, GW), axis=-1)
        gm_ref[pl.ds(j * GPS, GPS), :] = gmax.T

        @pl.when(j == NSTEP - 1)
        def _():
            gidv = lax.broadcasted_iota(jnp.int32, (NG, B), 0)
            klane = lax.broadcasted_iota(jnp.int32, (B, K), 1)

            def sel(k, carry):
                gm, topg = carry
                m = jnp.max(gm, axis=0, keepdims=True)
                gid = jnp.min(jnp.where(gm == m, gidv, BIGI), axis=0)
                topg = jnp.where(klane == k, gid[:, None], topg)
                gm = jnp.where(gidv == gid[None, :], -jnp.inf, gm)
                return gm, topg

            _, topg = lax.fori_loop(
                0, K, sel,
                (gm_ref[...], jnp.zeros((B, K), jnp.int32)))
            row = lax.broadcasted_iota(jnp.int32, (B, K), 0)
            topgf_ref[...] = topg + row * NG

    return pl.pallas_call(
        body,
        grid=(NSTEP,),
        in_specs=[
            pl.BlockSpec((B, DM), lambda j: (0, 0)),
            pl.BlockSpec((DM, VC), lambda j: (0, j)),
        ],
        out_specs=[
            pl.BlockSpec((B, VC), lambda j: (0, j)),
            pl.BlockSpec((B, K), lambda j: (0, 0)),
        ],
        out_shape=[
            jax.ShapeDtypeStruct((B, VPAD), jnp.float32),
            jax.ShapeDtypeStruct((B, K), jnp.int32),
        ],
        scratch_shapes=[pltpu.VMEM((NG, B), jnp.float32)],
    )(x, head)


def _finalize(cand, topgf, gnoise):
    C = K * GW
    PH = 10

    def body(cand_ref, topgf_ref, g_ref, next_ref, probs_ref, topi_ref,
             cs_ref):
        tgf = topgf_ref[...]
        rk = lax.broadcasted_iota(jnp.int32, (B, K, GW), 0)
        tg3 = tgf[:, :, None] - rk * NG
        vid3 = tg3 * GW + lax.broadcasted_iota(jnp.int32, (B, K, GW), 2)
        vid = vid3.reshape(B, C)
        cs_ref[...] = jnp.where(vid < VOCAB_N, cand_ref[...], -jnp.inf)
        klane = lax.broadcasted_iota(jnp.int32, (B, K), 1)

        tv = jnp.zeros((B, K), jnp.float32)
        ti = jnp.zeros((B, K), jnp.int32)
        for ph in range(K // PH):
            W = (ph + 1) * PH * GW
            vidp = vid[:, :W]

            def ext(k, carry, vidp=vidp):
                cp, tv, ti = carry
                m = jnp.max(cp, axis=1, keepdims=True)
                wv = jnp.min(jnp.where(cp == m, vidp, BIGI), axis=1)
                tv = jnp.where(klane == k, m, tv)
                ti = jnp.where(klane == k, wv[:, None], ti)
                cp = jnp.where(vidp == wv[:, None], -jnp.inf, cp)
                return cp, tv, ti

            cp, tv, ti = lax.fori_loop(
                ph * PH, (ph + 1) * PH, ext, (cs_ref[:, :W], tv, ti))
            if (ph + 1) * PH < K:
                cs_ref[:, :W] = cp

        z = tv
        p = jnp.exp(z - z[:, 0:1])
        probs = p / jnp.sum(p, axis=1, keepdims=True)
        probs_ref[...] = probs
        topi_ref[...] = ti
        score = jnp.log(probs + 1e-20) + g_ref[...]
        sm = jnp.max(score, axis=1, keepdims=True)
        ix = jnp.min(jnp.where(score == sm, klane, BIGI), axis=1)
        next_ref[...] = jnp.sum(
            jnp.where(klane == ix[:, None], ti, 0), axis=1)[:, None]

    return pl.pallas_call(
        body,
        out_shape=[
            jax.ShapeDtypeStruct((B, 1), jnp.int32),
            jax.ShapeDtypeStruct((B, K), jnp.float32),
            jax.ShapeDtypeStruct((B, K), jnp.int32),
        ],
        scratch_shapes=[pltpu.VMEM((B, C), jnp.float32)],
    )(cand, topgf, gnoise)


def kernel(idx, embed, head):
    last = idx[:, -1].astype(jnp.int32)
    g = jax.random.gumbel(jax.random.key(42), (B, K), jnp.float32)
    x = _sc_gather_rows(embed, last, rows_per_worker=8, workers=8)
    logits, topgf = _matmul_select(x, head)
    cand = _sc_gather_rows(logits.reshape(B * NG, GW), topgf.reshape(B * K),
                           rows_per_worker=128, workers=25)
    nxt, probs, topi = _finalize(cand.reshape(B, K * GW), topgf, g)
    return nxt, probs, topi


# speedup vs baseline: 2.0824x; 1.0012x over previous
import functools

import jax
import jax.numpy as jnp
from jax import lax
from jax.experimental import pallas as pl
from jax.experimental.pallas import tpu as pltpu
from jax.experimental.pallas import tpu_sc as plsc

B = 64
DM = 1024
VOCAB_N = 100000
K = 50
GW = 128
VC = 2048
NSTEP = 49
VPAD = NSTEP * VC
NG = VPAD // GW
GPS = VC // GW
BIGI = 2**30


def _sc_gather_rows(table, idxs, rows_per_worker, workers):
    info = plsc.get_sparse_core_info()
    nc = info.num_cores
    n, w = idxs.shape[0], table.shape[1]
    assert n == rows_per_worker * workers and rows_per_worker % 8 == 0
    mesh = plsc.VectorSubcoreMesh(core_axis_name="c", subcore_axis_name="s")

    @functools.partial(
        pl.kernel,
        mesh=mesh,
        out_type=jax.ShapeDtypeStruct((n, w), jnp.float32),
        scratch_types=[
            pltpu.VMEM((rows_per_worker,), jnp.int32),
            pltpu.VMEM((rows_per_worker, w), jnp.float32),
            pltpu.SemaphoreType.DMA,
        ],
    )
    def k(table_hbm, idx_hbm, out_hbm, idx_v, rows_v, sem):
        wid = lax.axis_index("s") * nc + lax.axis_index("c")

        @pl.when(wid < workers)
        def _():
            base = wid * rows_per_worker
            pltpu.sync_copy(idx_hbm.at[pl.ds(base, rows_per_worker)], idx_v)
            pltpu.async_copy(table_hbm.at[idx_v], rows_v, sem).wait()
            pltpu.sync_copy(rows_v, out_hbm.at[pl.ds(base, rows_per_worker)])

    return k(table, idxs)


def _matmul_select(x, head):
    def body(x_ref, h_ref, logits_ref, topgf_ref, gm_ref):
        j = pl.program_id(0)
        lg = jnp.dot(x_ref[...], h_ref[...],
                     preferred_element_type=jnp.float32)
        logits_ref[...] = lg
        col = lax.broadcasted_iota(jnp.int32, (B, VC), 1) + j * VC
        lgm = jnp.where(col < VOCAB_N, lg, -jnp.inf)
        gmax = jnp.max(lgm.reshape(B, GPS, GW), axis=-1)
        gm_ref[pl.ds(j * GPS, GPS), :] = gmax.T

        @pl.when(j == NSTEP - 1)
        def _():
            gidv = lax.broadcasted_iota(jnp.int32, (NG, B), 0)
            klane = lax.broadcasted_iota(jnp.int32, (B, K), 1)

            def sel(k, carry):
                gm, topg = carry
                m = jnp.max(gm, axis=0, keepdims=True)
                gid = jnp.min(jnp.where(gm == m, gidv, BIGI), axis=0)
                topg = jnp.where(klane == k, gid[:, None], topg)
                gm = jnp.where(gidv == gid[None, :], -jnp.inf, gm)
                return gm, topg

            _, topg = lax.fori_loop(
                0, K, sel,
                (gm_ref[...], jnp.zeros((B, K), jnp.int32)))
            row = lax.broadcasted_iota(jnp.int32, (B, K), 0)
            topgf_ref[...] = topg + row * NG

    return pl.pallas_call(
        body,
        grid=(NSTEP,),
        in_specs=[
            pl.BlockSpec((B, DM), lambda j: (0, 0)),
            pl.BlockSpec((DM, VC), lambda j: (0, j)),
        ],
        out_specs=[
            pl.BlockSpec((B, VC), lambda j: (0, j)),
            pl.BlockSpec((B, K), lambda j: (0, 0)),
        ],
        out_shape=[
            jax.ShapeDtypeStruct((B, VPAD), jnp.float32),
            jax.ShapeDtypeStruct((B, K), jnp.int32),
        ],
        scratch_shapes=[pltpu.VMEM((NG, B), jnp.float32)],
    )(x, head)


def _finalize(cand, topgf, gnoise):
    C = K * GW
    PH = 5

    def body(cand_ref, topgf_ref, g_ref, next_ref, probs_ref, topi_ref,
             cs_ref):
        tgf = topgf_ref[...]
        rk = lax.broadcasted_iota(jnp.int32, (B, K, GW), 0)
        tg3 = tgf[:, :, None] - rk * NG
        vid3 = tg3 * GW + lax.broadcasted_iota(jnp.int32, (B, K, GW), 2)
        vid = vid3.reshape(B, C)
        cs_ref[...] = jnp.where(vid < VOCAB_N, cand_ref[...], -jnp.inf)
        klane = lax.broadcasted_iota(jnp.int32, (B, K), 1)

        tv = jnp.zeros((B, K), jnp.float32)
        ti = jnp.zeros((B, K), jnp.int32)
        for ph in range(K // PH):
            W = (ph + 1) * PH * GW
            vidp = vid[:, :W]

            def ext(k, carry, vidp=vidp):
                cp, tv, ti = carry
                m = jnp.max(cp, axis=1, keepdims=True)
                wv = jnp.min(jnp.where(cp == m, vidp, BIGI), axis=1)
                tv = jnp.where(klane == k, m, tv)
                ti = jnp.where(klane == k, wv[:, None], ti)
                cp = jnp.where(vidp == wv[:, None], -jnp.inf, cp)
                return cp, tv, ti

            cp, tv, ti = lax.fori_loop(
                ph * PH, (ph + 1) * PH, ext, (cs_ref[:, :W], tv, ti))
            if (ph + 1) * PH < K:
                cs_ref[:, :W] = cp

        z = tv
        p = jnp.exp(z - z[:, 0:1])
        probs = p / jnp.sum(p, axis=1, keepdims=True)
        probs_ref[...] = probs
        topi_ref[...] = ti
        score = jnp.log(probs + 1e-20) + g_ref[...]
        sm = jnp.max(score, axis=1, keepdims=True)
        ix = jnp.min(jnp.where(score == sm, klane, BIGI), axis=1)
        next_ref[...] = jnp.sum(
            jnp.where(klane == ix[:, None], ti, 0), axis=1)[:, None]

    return pl.pallas_call(
        body,
        out_shape=[
            jax.ShapeDtypeStruct((B, 1), jnp.int32),
            jax.ShapeDtypeStruct((B, K), jnp.float32),
            jax.ShapeDtypeStruct((B, K), jnp.int32),
        ],
        scratch_shapes=[pltpu.VMEM((B, C), jnp.float32)],
    )(cand, topgf, gnoise)


def kernel(idx, embed, head):
    last = idx[:, -1].astype(jnp.int32)
    g = jax.random.gumbel(jax.random.key(42), (B, K), jnp.float32)
    x = _sc_gather_rows(embed, last, rows_per_worker=8, workers=8)
    logits, topgf = _matmul_select(x, head)
    cand = _sc_gather_rows(logits.reshape(B * NG, GW), topgf.reshape(B * K),
                           rows_per_worker=128, workers=25)
    nxt, probs, topi = _finalize(cand.reshape(B, K * GW), topgf, g)
    return nxt, probs, topi
